# baseline (speedup 1.0000x reference)
import functools

import jax
import jax.numpy as jnp
from jax import lax
from jax.experimental import pallas as pl
from jax.experimental.pallas import tpu as pltpu
from jax.experimental.pallas import tpu_sc as plsc

K = 256
N = 8192
R = 128
NC = 2
NS = 16
NW = NC * NS
ROWS_PER_W = R // NW
NCHUNK = N // 16

_SHIFTS = (23, 15, 7, 0)
_MASKS = (255, 255, 255, 127)
_NBINS = (256, 256, 256, 128)

_GDN = lax.GatherDimensionNumbers(
    offset_dims=(), collapsed_slice_dims=(0,), start_index_map=(0,))


def _vec(v):
    v = jnp.asarray(v)
    return v if v.ndim else jnp.broadcast_to(v, (16,))


def _splat_lane(vec, lane):
    idx = jnp.full((16, 1), lane, jnp.int32)
    return lax.gather(vec, idx, _GDN, (1,),
                      mode=lax.GatherScatterMode.PROMISE_IN_BOUNDS)


def _sc_body(x_hbm, vals_hbm, idx_hbm, row_ref, u_ref, ca_ref, cb_ref,
             hist_ref, vo_ref, io_ref, tmp_ref):
    wid = lax.axis_index("s") * NC + lax.axis_index("c")
    lanes = lax.iota(jnp.int32, 16)
    ones16 = jnp.ones((16,), jnp.int32)
    zeros16 = jnp.zeros((16,), jnp.int32)

    def zero_hist(nbins):
        for ci in range(nbins // 16):
            hist_ref[pl.ds(ci * 16, 16)] = zeros16

    def bin_scan(k_rem, nbins):
        done = zeros16
        ci_hit = zeros16
        f_hit = jnp.full((16,), 15, jnp.int32)
        s_hit = zeros16
        s = zeros16
        for i in range(nbins // 16):
            ci = nbins // 16 - 1 - i
            h = hist_ref[pl.ds(ci * 16, 16)]
            rcs = plsc.cumsum(lax.rev(h, (0,))) + s
            ge = rcs >= k_rem
            pc = _vec(plsc.all_reduce_population_count(ge))
            f = _vec(plsc.all_reduce_ffs(ge))
            hit = (done == 0) & (pc > 0)
            ci_hit = jnp.where(hit, ci, ci_hit)
            f_hit = jnp.where(hit, f, f_hit)
            s_hit = jnp.where(hit, s, s_hit)
            done = jnp.where(pc > 0, 1, done)
            s = _splat_lane(rcs, 15)
        ci_s = ci_hit[0]
        h = hist_ref[pl.ds(ci_s * 16, 16)]
        rcs = plsc.cumsum(lax.rev(h, (0,))) + s_hit
        tmp_ref[...] = rcs
        cnt_ge = plsc.load_gather(tmp_ref, [f_hit])
        b_star = ci_hit * 16 + 15 - f_hit
        hb = plsc.load_gather(hist_ref, [b_star])
        return b_star, cnt_ge - hb

    def row_body(r, _):
        row = wid * ROWS_PER_W + r
        pltpu.sync_copy(x_hbm.at[row], row_ref)

        zero_hist(_NBINS[0])

        def sweep0(g, _):
            for q in range(4):
                off = (g * 4 + q) * 16
                v = row_ref[pl.ds(off, 16)]
                u = jnp.maximum(plsc.bitcast(jnp.maximum(v, 0.0), jnp.int32), 0)
                u_ref[pl.ds(off, 16)] = u
                b = (u >> _SHIFTS[0]) & _MASKS[0]
                plsc.addupdate_scatter(hist_ref, [b], ones16)
            return 0

        lax.fori_loop(0, NCHUNK // 4, sweep0, 0)

        k_rem = jnp.full((16,), K, jnp.int32)
        b0, n_above = bin_scan(k_rem, _NBINS[0])
        k_rem = k_rem - n_above
        t_val = b0 << _SHIFTS[0]

        def compact0(g, off):
            for q in range(4):
                t = g * 4 + q
                u = u_ref[pl.ds(t * 16, 16)]
                sel = ((u >> _SHIFTS[0]) & _MASKS[0]) == b0
                tgt = off + plsc.cumsum(jnp.where(sel, 1, 0)) - 1
                plsc.store_scatter(ca_ref, [tgt], u, mask=sel)
                off = off + _vec(plsc.all_reduce_population_count(sel))
            return off

        nc_vec = lax.fori_loop(0, NCHUNK // 4, compact0, zeros16)

        src, dst = ca_ref, cb_ref
        for rnd in (1, 2, 3):
            sh = _SHIFTS[rnd]
            mk = _MASKS[rnd]
            zero_hist(_NBINS[rnd])
            nch = (nc_vec[0] + 15) // 16

            def hsweep(t, toff, src=src, sh=sh, mk=mk, nc=nc_vec):
                u = src[pl.ds(t * 16, 16)]
                valid = (lanes + toff) < nc
                b = (u >> sh) & mk
                plsc.addupdate_scatter(hist_ref, [b], ones16, mask=valid)
                return toff + 16

            lax.fori_loop(0, nch, hsweep, zeros16)
            br, n_above = bin_scan(k_rem, _NBINS[rnd])
            k_rem = k_rem - n_above
            t_val = t_val | (br << sh)

            if rnd < 3:
                def compact(t, carry, src=src, dst=dst, sh=sh, mk=mk,
                            nc=nc_vec, br=br):
                    off, toff = carry
                    u = src[pl.ds(t * 16, 16)]
                    valid = (lanes + toff) < nc
                    sel = valid & (((u >> sh) & mk) == br)
                    tgt = off + plsc.cumsum(jnp.where(sel, 1, 0)) - 1
                    plsc.store_scatter(dst, [tgt], u, mask=sel)
                    return off + _vec(plsc.all_reduce_population_count(sel)), \
                        toff + 16

                nc_vec, _ = lax.fori_loop(0, nch, compact, (zeros16, zeros16))
                src, dst = dst, src

        def fsweep(g, carry):
            off, budget, base = carry
            for q in range(4):
                t = g * 4 + q
                u = u_ref[pl.ds(t * 16, 16)]
                gt = u > t_val
                eq = u == t_val
                ngt = _vec(plsc.all_reduce_population_count(gt))
                neq = _vec(plsc.all_reduce_population_count(eq))
                take_eq = jnp.minimum(neq, budget)
                eqcs = plsc.cumsum(jnp.where(eq, 1, 0))
                sel = gt | (eq & (eqcs <= budget))
                tgt = off + plsc.cumsum(jnp.where(sel, 1, 0)) - 1
                plsc.store_scatter(vo_ref, [tgt], plsc.bitcast(u, jnp.float32),
                                   mask=sel)
                plsc.store_scatter(io_ref, [tgt], base + lanes, mask=sel)
                off = off + ngt + take_eq
                budget = budget - take_eq
                base = base + 16
            return off, budget, base

        lax.fori_loop(0, NCHUNK // 4, fsweep, (zeros16, k_rem, zeros16))

        pltpu.sync_copy(vo_ref, vals_hbm.at[row])
        pltpu.sync_copy(io_ref, idx_hbm.at[row])
        return 0

    lax.fori_loop(0, ROWS_PER_W, row_body, 0)


_sc_select = functools.partial(
    pl.kernel,
    out_type=[
        jax.ShapeDtypeStruct((R, K), jnp.float32),
        jax.ShapeDtypeStruct((R, K), jnp.int32),
    ],
    mesh=plsc.VectorSubcoreMesh(core_axis_name="c", subcore_axis_name="s"),
    compiler_params=pltpu.CompilerParams(needs_layout_passes=False),
    scratch_types=[
        pltpu.VMEM((N,), jnp.float32),
        pltpu.VMEM((N,), jnp.int32),
        pltpu.VMEM((N,), jnp.int32),
        pltpu.VMEM((N,), jnp.int32),
        pltpu.VMEM((256,), jnp.int32),
        pltpu.VMEM((K,), jnp.float32),
        pltpu.VMEM((K,), jnp.int32),
        pltpu.VMEM((16,), jnp.int32),
    ],
)(_sc_body)


def _sort_pairs_kernel(v_ref, i_ref, vo_ref, io_ref):
    v = v_ref[...]
    idx = i_ref[...]
    _, n = v.shape
    col = lax.broadcasted_iota(jnp.int32, v.shape, 1)
    k = 2
    while k <= n:
        j = k // 2
        while j >= 1:
            bit = (col & j) != 0
            pv = jnp.where(bit, jnp.roll(v, j, 1), jnp.roll(v, -j, 1))
            pi = jnp.where(bit, jnp.roll(idx, j, 1), jnp.roll(idx, -j, 1))
            own_better = (v > pv) | ((v == pv) & (idx < pi))
            block_up = (col & k) == 0
            i_lo = ~bit
            wants_better = i_lo == block_up
            keep = own_better == wants_better
            v = jnp.where(keep, v, pv)
            idx = jnp.where(keep, idx, pi)
            j //= 2
        k *= 2
    vo_ref[...] = v
    io_ref[...] = idx


def kernel(x):
    vals_u, idx_u = _sc_select(x)
    vals, idx = pl.pallas_call(
        _sort_pairs_kernel,
        out_shape=[
            jax.ShapeDtypeStruct((R, K), jnp.float32),
            jax.ShapeDtypeStruct((R, K), jnp.int32),
        ],
    )(vals_u, idx_u)
    return vals, idx


# speedup vs baseline: 1.1528x; 1.1528x over previous
import functools

import jax
import jax.numpy as jnp
from jax import lax
from jax.experimental import pallas as pl
from jax.experimental.pallas import tpu as pltpu
from jax.experimental.pallas import tpu_sc as plsc

K = 256
N = 8192
R = 128
NC = 2
NS = 16
NW = NC * NS
ROWS_PER_W = R // NW
NCHUNK = N // 16

_SHIFTS = (23, 15, 7, 0)
_MASKS = (255, 255, 255, 127)
_NBINS = (256, 256, 256, 128)

_GDN = lax.GatherDimensionNumbers(
    offset_dims=(), collapsed_slice_dims=(0,), start_index_map=(0,))


def _vec(v):
    v = jnp.asarray(v)
    return v if v.ndim else jnp.broadcast_to(v, (16,))


def _splat_lane(vec, lane):
    idx = jnp.full((16, 1), lane, jnp.int32)
    return lax.gather(vec, idx, _GDN, (1,),
                      mode=lax.GatherScatterMode.PROMISE_IN_BOUNDS)


def _sc_body(x_hbm, vals_hbm, idx_hbm, row_ref, u_ref, ca_ref, cb_ref,
             hist_ref, vo_ref, io_ref, tmp_ref):
    wid = lax.axis_index("s") * NC + lax.axis_index("c")
    lanes = lax.iota(jnp.int32, 16)
    ones16 = jnp.ones((16,), jnp.int32)
    zeros16 = jnp.zeros((16,), jnp.int32)

    def zero_hist(nbins):
        for ci in range(nbins // 16):
            hist_ref[pl.ds(ci * 16, 16)] = zeros16

    def bin_scan(k_rem, nbins):
        done = zeros16
        ci_hit = zeros16
        f_hit = jnp.full((16,), 15, jnp.int32)
        s_hit = zeros16
        s = zeros16
        for i in range(nbins // 16):
            ci = nbins // 16 - 1 - i
            h = hist_ref[pl.ds(ci * 16, 16)]
            rcs = plsc.cumsum(lax.rev(h, (0,))) + s
            ge = rcs >= k_rem
            pc = _vec(plsc.all_reduce_population_count(ge))
            f = _vec(plsc.all_reduce_ffs(ge))
            hit = (done == 0) & (pc > 0)
            ci_hit = jnp.where(hit, ci, ci_hit)
            f_hit = jnp.where(hit, f, f_hit)
            s_hit = jnp.where(hit, s, s_hit)
            done = jnp.where(pc > 0, 1, done)
            s = _splat_lane(rcs, 15)
        ci_s = ci_hit[0]
        h = hist_ref[pl.ds(ci_s * 16, 16)]
        rcs = plsc.cumsum(lax.rev(h, (0,))) + s_hit
        tmp_ref[...] = rcs
        cnt_ge = plsc.load_gather(tmp_ref, [f_hit])
        b_star = ci_hit * 16 + 15 - f_hit
        hb = plsc.load_gather(hist_ref, [b_star])
        return b_star, cnt_ge - hb

    def row_body(r, _):
        plsc.subcore_barrier()
        row = wid * ROWS_PER_W + r
        pltpu.sync_copy(x_hbm.at[row], row_ref)

        zero_hist(_NBINS[0])

        def sweep0(g, _):
            for q in range(4):
                off = (g * 4 + q) * 16
                v = row_ref[pl.ds(off, 16)]
                u = jnp.maximum(plsc.bitcast(jnp.maximum(v, 0.0), jnp.int32), 0)
                u_ref[pl.ds(off, 16)] = u
                b = (u >> _SHIFTS[0]) & _MASKS[0]
                plsc.addupdate_scatter(hist_ref, [b], ones16)
            return 0

        lax.fori_loop(0, NCHUNK // 4, sweep0, 0)

        k_rem = jnp.full((16,), K, jnp.int32)
        b0, n_above = bin_scan(k_rem, _NBINS[0])
        k_rem = k_rem - n_above
        t_val = b0 << _SHIFTS[0]

        def compact0(g, off):
            for q in range(4):
                t = g * 4 + q
                u = u_ref[pl.ds(t * 16, 16)]
                sel = ((u >> _SHIFTS[0]) & _MASKS[0]) == b0
                plsc.store_compressed(ca_ref.at[pl.ds(off, 16)], u, mask=sel)
                off = off + _vec(plsc.all_reduce_population_count(sel))[0]
            return off

        nc_cur = lax.fori_loop(0, NCHUNK // 4, compact0, jnp.int32(0))

        src, dst = ca_ref, cb_ref
        for rnd in (1, 2, 3):
            sh = _SHIFTS[rnd]
            mk = _MASKS[rnd]
            zero_hist(_NBINS[rnd])
            nch = (nc_cur + 15) // 16
            nc_vec = jnp.broadcast_to(nc_cur, (16,))

            def hsweep(t, toff, src=src, sh=sh, mk=mk, nc=nc_vec):
                u = src[pl.ds(t * 16, 16)]
                valid = (lanes + toff) < nc
                b = (u >> sh) & mk
                plsc.addupdate_scatter(hist_ref, [b], ones16, mask=valid)
                return toff + 16

            lax.fori_loop(0, nch, hsweep, zeros16)
            br, n_above = bin_scan(k_rem, _NBINS[rnd])
            k_rem = k_rem - n_above
            t_val = t_val | (br << sh)

            if rnd < 3:
                def compact(t, carry, src=src, dst=dst, sh=sh, mk=mk,
                            nc=nc_vec, br=br):
                    off, toff = carry
                    u = src[pl.ds(t * 16, 16)]
                    valid = (lanes + toff) < nc
                    sel = valid & (((u >> sh) & mk) == br)
                    plsc.store_compressed(dst.at[pl.ds(off, 16)], u, mask=sel)
                    return off + _vec(plsc.all_reduce_population_count(sel))[0], \
                        toff + 16

                nc_cur, _ = lax.fori_loop(0, nch, compact,
                                          (jnp.int32(0), zeros16))
                src, dst = dst, src

        plsc.subcore_barrier()
        budget0 = k_rem[0]

        def fsweep(g, carry):
            off, budget = carry
            for q in range(4):
                t = g * 4 + q
                u = u_ref[pl.ds(t * 16, 16)]
                gt = u > t_val
                eq = u == t_val
                ngt = _vec(plsc.all_reduce_population_count(gt))[0]
                neq = _vec(plsc.all_reduce_population_count(eq))[0]
                take_eq = jnp.minimum(neq, budget)
                eqcs = plsc.cumsum(jnp.where(eq, 1, 0))
                sel = gt | (eq & (eqcs <= budget))
                vals = plsc.bitcast(u, jnp.float32)
                idxv = t * 16 + lanes
                plsc.store_compressed(vo_ref.at[pl.ds(off, 16)], vals, mask=sel)
                plsc.store_compressed(io_ref.at[pl.ds(off, 16)], idxv, mask=sel)
                off = off + ngt + take_eq
                budget = budget - take_eq
            return off, budget

        lax.fori_loop(0, NCHUNK // 4, fsweep, (jnp.int32(0), budget0))

        pltpu.sync_copy(vo_ref, vals_hbm.at[row])
        pltpu.sync_copy(io_ref, idx_hbm.at[row])
        return 0

    lax.fori_loop(0, ROWS_PER_W, row_body, 0)


_sc_select = functools.partial(
    pl.kernel,
    out_type=[
        jax.ShapeDtypeStruct((R, K), jnp.float32),
        jax.ShapeDtypeStruct((R, K), jnp.int32),
    ],
    mesh=plsc.VectorSubcoreMesh(core_axis_name="c", subcore_axis_name="s"),
    compiler_params=pltpu.CompilerParams(needs_layout_passes=False),
    scratch_types=[
        pltpu.VMEM((N,), jnp.float32),
        pltpu.VMEM((N,), jnp.int32),
        pltpu.VMEM((N,), jnp.int32),
        pltpu.VMEM((N,), jnp.int32),
        pltpu.VMEM((256,), jnp.int32),
        pltpu.VMEM((K,), jnp.float32),
        pltpu.VMEM((K,), jnp.int32),
        pltpu.VMEM((16,), jnp.int32),
    ],
)(_sc_body)


def _sort_pairs_kernel(v_ref, i_ref, vo_ref, io_ref):
    v = v_ref[...]
    idx = i_ref[...]
    _, n = v.shape
    col = lax.broadcasted_iota(jnp.int32, v.shape, 1)
    k = 2
    while k <= n:
        j = k // 2
        while j >= 1:
            bit = (col & j) != 0
            pv = jnp.where(bit, jnp.roll(v, j, 1), jnp.roll(v, -j, 1))
            pi = jnp.where(bit, jnp.roll(idx, j, 1), jnp.roll(idx, -j, 1))
            own_better = (v > pv) | ((v == pv) & (idx < pi))
            block_up = (col & k) == 0
            i_lo = ~bit
            wants_better = i_lo == block_up
            keep = own_better == wants_better
            v = jnp.where(keep, v, pv)
            idx = jnp.where(keep, idx, pi)
            j //= 2
        k *= 2
    vo_ref[...] = v
    io_ref[...] = idx


def kernel(x):
    vals_u, idx_u = _sc_select(x)
    vals, idx = pl.pallas_call(
        _sort_pairs_kernel,
        out_shape=[
            jax.ShapeDtypeStruct((R, K), jnp.float32),
            jax.ShapeDtypeStruct((R, K), jnp.int32),
        ],
    )(vals_u, idx_u)
    return vals, idx
